# Initial kernel scaffold; baseline (speedup 1.0000x reference)
import jax
import jax.numpy as jnp
from jax.experimental import pallas as pl


def kernel(x, y, W, b, table):
    raise NotImplementedError("write your pallas kernel here")


# speedup vs baseline: 2.5167x; 2.5167x over previous
import functools

import jax
import jax.numpy as jnp
from jax.experimental import pallas as pl


def _tokenizer_kernel(x_ref, y_ref, w_ref, b_ref, t_ref, o_ref, *, n_classes):
    x = x_ref[0]
    y = y_ref[0]
    proj = jax.lax.dot_general(
        x, w_ref[...],
        dimension_numbers=(((1,), (1,)), ((), ())),
        preferred_element_type=jnp.float32,
    )
    classes = jax.lax.broadcasted_iota(jnp.int32, (1, n_classes), 1)
    onehot = (y == classes).astype(jnp.float32)
    lab = jax.lax.dot_general(
        onehot, t_ref[...],
        dimension_numbers=(((1,), (0,)), ((), ())),
        preferred_element_type=jnp.float32,
    )
    o_ref[0] = proj + lab + b_ref[...]


@jax.jit
def kernel(x, y, W, b, table):
    B, N, D = x.shape
    H, _ = W.shape
    C = table.shape[0]
    rows = B * N
    R = 4096
    G = rows // R

    xf = x.reshape(G, R, D)
    y3 = y.reshape(G, R, 1)
    b2 = b.reshape(1, H)

    out = pl.pallas_call(
        functools.partial(_tokenizer_kernel, n_classes=C),
        grid=(G,),
        in_specs=[
            pl.BlockSpec((1, R, D), lambda i: (i, 0, 0)),
            pl.BlockSpec((1, R, 1), lambda i: (i, 0, 0)),
            pl.BlockSpec((H, D), lambda i: (0, 0)),
            pl.BlockSpec((1, H), lambda i: (0, 0)),
            pl.BlockSpec((C, H), lambda i: (0, 0)),
        ],
        out_specs=pl.BlockSpec((1, R, H), lambda i: (i, 0, 0)),
        out_shape=jax.ShapeDtypeStruct((G, R, H), jnp.float32),
    )(xf, y3, W, b2, table)
    return out.reshape(B, N, H)


# speedup vs baseline: 3.8048x; 1.5118x over previous
import functools

import jax
import jax.numpy as jnp
from jax.experimental import pallas as pl


def _tokenizer_kernel(x_ref, y_ref, w_ref, t_ref, o_ref, *, n_classes, bb):
    classes = jax.lax.broadcasted_iota(jnp.int32, (1, n_classes), 1)
    for j in range(bb):
        xj = x_ref[j]
        yj = y_ref[j]
        proj = jax.lax.dot_general(
            xj, w_ref[...],
            dimension_numbers=(((1,), (1,)), ((), ())),
            preferred_element_type=jnp.float32,
        )
        onehot = (yj[:, None] == classes).astype(jnp.float32)
        lab = jax.lax.dot_general(
            onehot, t_ref[...],
            dimension_numbers=(((1,), (0,)), ((), ())),
            preferred_element_type=jnp.float32,
        )
        o_ref[j] = proj + lab


@jax.jit
def kernel(x, y, W, b, table):
    B, N, D = x.shape
    H, _ = W.shape
    C = table.shape[0]
    BB = 8
    G = B // BB

    table_b = table + b[None, :]

    out = pl.pallas_call(
        functools.partial(_tokenizer_kernel, n_classes=C, bb=BB),
        grid=(G,),
        in_specs=[
            pl.BlockSpec((BB, N, D), lambda i: (i, 0, 0)),
            pl.BlockSpec((BB, N), lambda i: (i, 0)),
            pl.BlockSpec((H, D), lambda i: (0, 0)),
            pl.BlockSpec((C, H), lambda i: (0, 0)),
        ],
        out_specs=pl.BlockSpec((BB, N, H), lambda i: (i, 0, 0)),
        out_shape=jax.ShapeDtypeStruct((B, N, H), jnp.float32),
    )(x, y, W, table_b)
    return out


# speedup vs baseline: 4.3825x; 1.1519x over previous
import functools

import jax
import jax.numpy as jnp
import jax.scipy.linalg as jsl
from jax.experimental import pallas as pl


def _tokenizer_kernel(x_ref, y_ref, w_ref, t_ref, o_ref, *, bb):
    classes = jax.lax.broadcasted_iota(jnp.int32, (1, 128), 1)
    for jb in range(bb):
        xj = x_ref[jb]
        acc = jax.lax.dot_general(
            xj, w_ref[...],
            dimension_numbers=(((1,), (0,)), ((), ())),
            preferred_element_type=jnp.float32,
        )
        for j in range(4):
            yjs = y_ref[jb, j][:, None]
            onehot = (yjs == classes).astype(jnp.float32)
            acc += jax.lax.dot_general(
                onehot, t_ref[j],
                dimension_numbers=(((1,), (0,)), ((), ())),
                preferred_element_type=jnp.float32,
            )
        o_ref[jb] = acc


@jax.jit
def kernel(x, y, W, b, table):
    B, N, D = x.shape
    H, _ = W.shape
    C = table.shape[0]
    BB = 8
    G = B // BB

    xp = x.reshape(B, N // 4, 4 * D)
    ys = jnp.transpose(y.reshape(B, N // 4, 4), (0, 2, 1))
    table_b = table + b[None, :]
    Wt = W.T
    Wd = jsl.block_diag(Wt, Wt, Wt, Wt)
    T4 = jnp.zeros((4, 128, 4 * H), jnp.float32)
    for j in range(4):
        T4 = T4.at[j, :C, j * H:(j + 1) * H].set(table_b)

    out = pl.pallas_call(
        functools.partial(_tokenizer_kernel, bb=BB),
        grid=(G,),
        in_specs=[
            pl.BlockSpec((BB, N // 4, 4 * D), lambda i: (i, 0, 0)),
            pl.BlockSpec((BB, 4, N // 4), lambda i: (i, 0, 0)),
            pl.BlockSpec((4 * D, 4 * H), lambda i: (0, 0)),
            pl.BlockSpec((4, 128, 4 * H), lambda i: (0, 0, 0)),
        ],
        out_specs=pl.BlockSpec((BB, N // 4, 4 * H), lambda i: (i, 0, 0)),
        out_shape=jax.ShapeDtypeStruct((B, N // 4, 4 * H), jnp.float32),
    )(xp, ys, Wd, T4)
    return out.reshape(B, N, H)
